# baseline (speedup 1.0000x reference)
import jax
import jax.numpy as jnp
from jax.experimental import pallas as pl
from jax.experimental.pallas import tpu as pltpu

N = 10000
E = 160000
D = 256
HID = 256
OUT = 128
B = 8


def _gelu(x):
    c = 0.7978845608028654
    return 0.5 * x * (1.0 + jnp.tanh(c * (x + 0.044715 * x * x * x)))


def _head_body(p_ref, h1_ref, hb1_ref, h2_ref, hb2_ref, h3_ref, hb3_ref, o_ref):
    p = p_ref[...]
    p = _gelu(jnp.dot(p, h1_ref[...], preferred_element_type=jnp.float32) + hb1_ref[...])
    p = _gelu(jnp.dot(p, h2_ref[...], preferred_element_type=jnp.float32) + hb2_ref[...])
    o_ref[...] = jnp.dot(p, h3_ref[...], preferred_element_type=jnp.float32) + hb3_ref[...]


def _head(p, H1, hb1, H2, hb2, H3, hb3):
    return pl.pallas_call(
        _head_body,
        out_shape=jax.ShapeDtypeStruct((B, OUT), jnp.float32),
    )(p, H1, hb1[None, :], H2, hb2[None, :], H3, hb3[None, :])


def kernel(x, edge_index, batch, W1, b1, g1, be1, W2, b2, W3, b3, g3, be3, W4, b4, H1, hb1, H2, hb2, H3, hb3):
    n = x.shape[0]
    loop = jnp.arange(n, dtype=edge_index.dtype)
    src = jnp.concatenate([edge_index[0], loop])
    dst = jnp.concatenate([edge_index[1], loop])

    deg = jax.ops.segment_sum(jnp.ones_like(src, dtype=x.dtype), dst, num_segments=n)
    dinv = jnp.where(deg > 0, deg ** -0.5, 0.0)
    coef = dinv[src] * dinv[dst]

    def conv(h, W, b):
        hw = h @ W
        msg = hw[src] * coef[:, None]
        return jax.ops.segment_sum(msg, dst, num_segments=n) + b

    def bn(h, g, b):
        m = jnp.mean(h, axis=0)
        v = jnp.var(h, axis=0)
        return (h - m) / jnp.sqrt(v + 1e-5) * g + b

    h = conv(x, W1, b1)
    h = jax.nn.gelu(bn(h, g1, be1))
    h = conv(h, W2, b2)
    h = conv(h, W3, b3)
    h = jax.nn.gelu(bn(h, g3, be3))
    h = conv(h, W4, b4)

    ssum = jax.ops.segment_sum(h, batch, num_segments=B)
    cnt = jax.ops.segment_sum(jnp.ones((n,), h.dtype), batch, num_segments=B)
    p = ssum / jnp.maximum(cnt, 1.0)[:, None]
    return _head(p, H1, hb1, H2, hb2, H3, hb3)


# speedup vs baseline: 2.2202x; 2.2202x over previous
import functools

import jax
import jax.numpy as jnp
from jax import lax
from jax.experimental import pallas as pl
from jax.experimental.pallas import tpu as pltpu
from jax.experimental.pallas import tpu_sc as plsc

N = 10000
E = 160000
D = 256
HID = 256
OUT = 128
B = 8

NC = 2
NS = 16
DH = 128
N_PAD = 10240
RPT = N_PAD // NS

CH = 16
EPT = E // NS
NCHUNK = EPT // CH

CH_DEG = 40
EPT_DEG = E // (NC * NS)
NCHUNK_DEG = EPT_DEG // CH_DEG

_MESH = plsc.VectorSubcoreMesh(core_axis_name="c", subcore_axis_name="s")


def _agg_body(h_hbm, src_hbm, dst_hbm, out_hbm, isrc, idst, rows, acc, sem_i, sem_g, sem_s):
    c = lax.axis_index("c")
    s = lax.axis_index("s")

    pltpu.sync_copy(h_hbm.at[c, pl.ds(s * RPT, RPT)], acc.at[pl.ds(s * RPT, RPT)])
    plsc.subcore_barrier()

    base = s * EPT

    def chunk(g, buf):
        e0 = base + g * CH
        pltpu.async_copy(src_hbm.at[pl.ds(e0, CH)], isrc.at[buf], sem_i).wait()
        pltpu.async_copy(dst_hbm.at[pl.ds(e0, CH)], idst.at[buf], sem_i).wait()
        pltpu.async_copy(h_hbm.at[c].at[isrc.at[buf]], rows.at[buf], sem_g).wait()
        pltpu.async_copy(rows.at[buf], acc.at[idst.at[buf]], sem_s, add=True).wait()

    def body(i, carry):
        chunk(2 * i, 0)
        chunk(2 * i + 1, 1)
        return carry

    lax.fori_loop(0, NCHUNK // 2, body, 0)

    plsc.subcore_barrier()
    pltpu.sync_copy(acc.at[pl.ds(s * RPT, RPT)], out_hbm.at[c, pl.ds(s * RPT, RPT)])


def _sc_aggregate(h_split, src, dst):
    k = functools.partial(
        pl.kernel,
        out_type=jax.ShapeDtypeStruct((NC, N_PAD, DH), jnp.float32),
        mesh=_MESH,
        scratch_types=[
            pltpu.VMEM((2, CH), jnp.int32),
            pltpu.VMEM((2, CH), jnp.int32),
            pltpu.VMEM((2, CH, DH), jnp.float32),
            pltpu.VMEM_SHARED((N_PAD, DH), jnp.float32),
            pltpu.SemaphoreType.DMA,
            pltpu.SemaphoreType.DMA,
            pltpu.SemaphoreType.DMA,
        ],
    )(_agg_body)
    return k(h_split, src, dst)


def _deg_body(dst_hbm, zeros_hbm, ones_hbm, out_hbm, ibuf, ones_v, dacc, sem):
    c = lax.axis_index("c")
    s = lax.axis_index("s")

    pltpu.sync_copy(zeros_hbm.at[pl.ds(s * RPT, RPT)], dacc.at[pl.ds(s * RPT, RPT)])
    pltpu.sync_copy(ones_hbm, ones_v)
    plsc.subcore_barrier()

    base = c * (E // NC) + s * EPT_DEG

    def body(g, carry):
        pltpu.async_copy(dst_hbm.at[pl.ds(base + g * CH_DEG, CH_DEG)], ibuf, sem).wait()
        pltpu.sync_copy(ones_v.at[pl.ds(0, CH_DEG)], dacc.at[ibuf], add=True)
        return carry

    lax.fori_loop(0, NCHUNK_DEG, body, 0)

    plsc.subcore_barrier()
    pltpu.sync_copy(dacc.at[pl.ds(s * RPT, RPT)], out_hbm.at[c, pl.ds(s * RPT, RPT)])


def _sc_degree(dst):
    k = functools.partial(
        pl.kernel,
        out_type=jax.ShapeDtypeStruct((NC, N_PAD), jnp.float32),
        mesh=_MESH,
        scratch_types=[
            pltpu.VMEM((CH_DEG,), jnp.int32),
            pltpu.VMEM((128,), jnp.float32),
            pltpu.VMEM_SHARED((N_PAD,), jnp.float32),
            pltpu.SemaphoreType.DMA,
        ],
    )(_deg_body)
    return k(dst, jnp.zeros((N_PAD,), jnp.float32), jnp.ones((128,), jnp.float32))


def _gelu(x):
    c = 0.7978845608028654
    return 0.5 * x * (1.0 + jnp.tanh(c * (x + 0.044715 * x * x * x)))


def _head_body(p_ref, h1_ref, hb1_ref, h2_ref, hb2_ref, h3_ref, hb3_ref, o_ref):
    p = p_ref[...]
    p = _gelu(jnp.dot(p, h1_ref[...], preferred_element_type=jnp.float32) + hb1_ref[...])
    p = _gelu(jnp.dot(p, h2_ref[...], preferred_element_type=jnp.float32) + hb2_ref[...])
    o_ref[...] = jnp.dot(p, h3_ref[...], preferred_element_type=jnp.float32) + hb3_ref[...]


def _head(p, H1, hb1, H2, hb2, H3, hb3):
    return pl.pallas_call(
        _head_body,
        out_shape=jax.ShapeDtypeStruct((B, OUT), jnp.float32),
    )(p, H1, hb1[None, :], H2, hb2[None, :], H3, hb3[None, :])


def kernel(x, edge_index, batch, W1, b1, g1, be1, W2, b2, W3, b3, g3, be3, W4, b4, H1, hb1, H2, hb2, H3, hb3):
    deg_halves = _sc_degree(edge_index[1])
    deg = deg_halves[0, :N] + deg_halves[1, :N] + 1.0
    dinv = deg ** -0.5

    src, dst = edge_index[0], edge_index[1]

    def conv(h, W, b):
        hp = (h @ W) * dinv[:, None]
        hp_split = jnp.stack([hp[:, :DH], hp[:, DH:]])
        hp_split = jnp.pad(hp_split, ((0, 0), (0, N_PAD - N), (0, 0)))
        agg = _sc_aggregate(hp_split, src, dst)
        agg = jnp.concatenate([agg[0, :N], agg[1, :N]], axis=1)
        return agg * dinv[:, None] + b

    def bn(h, g, b):
        m = jnp.mean(h, axis=0)
        v = jnp.var(h, axis=0)
        return (h - m) / jnp.sqrt(v + 1e-5) * g + b

    h = conv(x, W1, b1)
    h = jax.nn.gelu(bn(h, g1, be1))
    h = conv(h, W2, b2)
    h = conv(h, W3, b3)
    h = jax.nn.gelu(bn(h, g3, be3))
    h = conv(h, W4, b4)

    ssum = jax.ops.segment_sum(h, batch, num_segments=B)
    cnt = jax.ops.segment_sum(jnp.ones((N,), h.dtype), batch, num_segments=B)
    p = ssum / jnp.maximum(cnt, 1.0)[:, None]
    return _head(p, H1, hb1, H2, hb2, H3, hb3)


# speedup vs baseline: 8.4976x; 3.8274x over previous
import functools

import jax
import jax.numpy as jnp
from jax import lax
from jax.experimental import pallas as pl
from jax.experimental.pallas import tpu as pltpu
from jax.experimental.pallas import tpu_sc as plsc

N = 10000
E = 160000
D = 256
HID = 256
OUT = 128
B = 8

NC = 2
NS = 16
DH = 128
N_PAD = 10240
RPT = N_PAD // NS

CH = 80
NSUB = CH // 16
EPT = E // NS
NCHUNK = EPT // CH

CH_DEG = 40
EPT_DEG = E // (NC * NS)
NCHUNK_DEG = EPT_DEG // CH_DEG

_MESH = plsc.VectorSubcoreMesh(core_axis_name="c", subcore_axis_name="s")


def _agg_body(h_hbm, src_hbm, dst_hbm, out_hbm,
              isrcA, isrcB, idstA, idstB, rowsA, rowsB, acc,
              semIA, semIB, semGA, semGB, semSA, semSB):
    c = lax.axis_index("c")
    s = lax.axis_index("s")
    base = s * EPT

    pltpu.sync_copy(h_hbm.at[c, pl.ds(s * RPT, RPT)], acc.at[pl.ds(s * RPT, RPT)])
    plsc.subcore_barrier()

    def issue_idx(g, isrc, idst, semI):
        e0 = base + g * CH
        pltpu.async_copy(src_hbm.at[pl.ds(e0, CH)], isrc, semI)
        for j in range(NSUB):
            pltpu.async_copy(dst_hbm.at[pl.ds(e0 + 16 * j, 16)], idst.at[j], semI)

    def wait_idx(g, isrc, idst, semI):
        e0 = base + g * CH
        pltpu.make_async_copy(src_hbm.at[pl.ds(e0, CH)], isrc, semI).wait()
        for j in range(NSUB):
            pltpu.make_async_copy(dst_hbm.at[pl.ds(e0 + 16 * j, 16)], idst.at[j], semI).wait()

    def issue_gather(isrc, rows, semG):
        pltpu.async_copy(h_hbm.at[c].at[isrc], rows, semG)

    def wait_gather(isrc, rows, semG):
        pltpu.make_async_copy(h_hbm.at[c].at[isrc], rows, semG).wait()

    def issue_scat(idst, rows, semS):
        for j in range(NSUB):
            pltpu.async_copy(rows.at[pl.ds(16 * j, 16)], acc.at[idst.at[j]], semS, add=True)

    def wait_scat(idst, rows, semS):
        for j in range(NSUB):
            pltpu.make_async_copy(rows.at[pl.ds(16 * j, 16)], acc.at[idst.at[j]], semS).wait()

    A = (isrcA, idstA, rowsA, semIA, semGA, semSA)
    Bv = (isrcB, idstB, rowsB, semIB, semGB, semSB)

    issue_idx(0, isrcA, idstA, semIA)
    wait_idx(0, isrcA, idstA, semIA)
    issue_gather(isrcA, rowsA, semGA)

    wait_gather(isrcA, rowsA, semGA)
    issue_scat(idstA, rowsA, semSA)
    issue_idx(1, isrcB, idstB, semIB)
    wait_idx(1, isrcB, idstB, semIB)
    issue_gather(isrcB, rowsB, semGB)
    wait_gather(isrcB, rowsB, semGB)
    issue_scat(idstB, rowsB, semSB)
    wait_scat(idstA, rowsA, semSA)
    issue_idx(2, isrcA, idstA, semIA)
    wait_idx(2, isrcA, idstA, semIA)
    issue_gather(isrcA, rowsA, semGA)

    def body(i, carry):
        g = 2 * i
        wait_gather(isrcA, rowsA, semGA)
        issue_scat(idstA, rowsA, semSA)
        wait_scat(idstB, rowsB, semSB)
        issue_idx(g + 1, isrcB, idstB, semIB)
        wait_idx(g + 1, isrcB, idstB, semIB)
        issue_gather(isrcB, rowsB, semGB)
        wait_gather(isrcB, rowsB, semGB)
        issue_scat(idstB, rowsB, semSB)
        wait_scat(idstA, rowsA, semSA)
        issue_idx(g + 2, isrcA, idstA, semIA)
        wait_idx(g + 2, isrcA, idstA, semIA)
        issue_gather(isrcA, rowsA, semGA)
        return carry

    lax.fori_loop(1, NCHUNK // 2, body, 0)

    wait_gather(isrcA, rowsA, semGA)
    issue_scat(idstA, rowsA, semSA)
    wait_scat(idstB, rowsB, semSB)
    wait_scat(idstA, rowsA, semSA)

    plsc.subcore_barrier()
    pltpu.sync_copy(acc.at[pl.ds(s * RPT, RPT)], out_hbm.at[c, pl.ds(s * RPT, RPT)])


def _sc_aggregate(h_split, src, dst):
    k = functools.partial(
        pl.kernel,
        out_type=jax.ShapeDtypeStruct((NC, N_PAD, DH), jnp.float32),
        mesh=_MESH,
        scratch_types=[
            pltpu.VMEM((CH,), jnp.int32),
            pltpu.VMEM((CH,), jnp.int32),
            pltpu.VMEM((NSUB, 16), jnp.int32),
            pltpu.VMEM((NSUB, 16), jnp.int32),
            pltpu.VMEM((CH, DH), jnp.float32),
            pltpu.VMEM((CH, DH), jnp.float32),
            pltpu.VMEM_SHARED((N_PAD, DH), jnp.float32),
            pltpu.SemaphoreType.DMA,
            pltpu.SemaphoreType.DMA,
            pltpu.SemaphoreType.DMA,
            pltpu.SemaphoreType.DMA,
            pltpu.SemaphoreType.DMA,
            pltpu.SemaphoreType.DMA,
        ],
    )(_agg_body)
    return k(h_split, src, dst)


def _deg_body(dst_hbm, zeros_hbm, ones_hbm, out_hbm, ibuf, ones_v, dacc, sem):
    c = lax.axis_index("c")
    s = lax.axis_index("s")

    pltpu.sync_copy(zeros_hbm.at[pl.ds(s * RPT, RPT)], dacc.at[pl.ds(s * RPT, RPT)])
    pltpu.sync_copy(ones_hbm, ones_v)
    plsc.subcore_barrier()

    base = c * (E // NC) + s * EPT_DEG

    def body(g, carry):
        pltpu.async_copy(dst_hbm.at[pl.ds(base + g * CH_DEG, CH_DEG)], ibuf, sem).wait()
        pltpu.sync_copy(ones_v.at[pl.ds(0, CH_DEG)], dacc.at[ibuf], add=True)
        return carry

    lax.fori_loop(0, NCHUNK_DEG, body, 0)

    plsc.subcore_barrier()
    pltpu.sync_copy(dacc.at[pl.ds(s * RPT, RPT)], out_hbm.at[c, pl.ds(s * RPT, RPT)])


def _sc_degree(dst):
    k = functools.partial(
        pl.kernel,
        out_type=jax.ShapeDtypeStruct((NC, N_PAD), jnp.float32),
        mesh=_MESH,
        scratch_types=[
            pltpu.VMEM((CH_DEG,), jnp.int32),
            pltpu.VMEM((128,), jnp.float32),
            pltpu.VMEM_SHARED((N_PAD,), jnp.float32),
            pltpu.SemaphoreType.DMA,
        ],
    )(_deg_body)
    return k(dst, jnp.zeros((N_PAD,), jnp.float32), jnp.ones((128,), jnp.float32))


def _gelu(x):
    c = 0.7978845608028654
    return 0.5 * x * (1.0 + jnp.tanh(c * (x + 0.044715 * x * x * x)))


def _head_body(p_ref, h1_ref, hb1_ref, h2_ref, hb2_ref, h3_ref, hb3_ref, o_ref):
    p = p_ref[...]
    p = _gelu(jnp.dot(p, h1_ref[...], preferred_element_type=jnp.float32) + hb1_ref[...])
    p = _gelu(jnp.dot(p, h2_ref[...], preferred_element_type=jnp.float32) + hb2_ref[...])
    o_ref[...] = jnp.dot(p, h3_ref[...], preferred_element_type=jnp.float32) + hb3_ref[...]


def _head(p, H1, hb1, H2, hb2, H3, hb3):
    return pl.pallas_call(
        _head_body,
        out_shape=jax.ShapeDtypeStruct((B, OUT), jnp.float32),
    )(p, H1, hb1[None, :], H2, hb2[None, :], H3, hb3[None, :])


def kernel(x, edge_index, batch, W1, b1, g1, be1, W2, b2, W3, b3, g3, be3, W4, b4, H1, hb1, H2, hb2, H3, hb3):
    deg_halves = _sc_degree(edge_index[1])
    deg = deg_halves[0, :N] + deg_halves[1, :N] + 1.0
    dinv = deg ** -0.5

    src, dst = edge_index[0], edge_index[1]

    def conv(h, W, b):
        hp = (h @ W) * dinv[:, None]
        hp_split = jnp.stack([hp[:, :DH], hp[:, DH:]])
        hp_split = jnp.pad(hp_split, ((0, 0), (0, N_PAD - N), (0, 0)))
        agg = _sc_aggregate(hp_split, src, dst)
        agg = jnp.concatenate([agg[0, :N], agg[1, :N]], axis=1)
        return agg * dinv[:, None] + b

    def bn(h, g, b):
        m = jnp.mean(h, axis=0)
        v = jnp.var(h, axis=0)
        return (h - m) / jnp.sqrt(v + 1e-5) * g + b

    h = conv(x, W1, b1)
    h = jax.nn.gelu(bn(h, g1, be1))
    h = conv(h, W2, b2)
    h = conv(h, W3, b3)
    h = jax.nn.gelu(bn(h, g3, be3))
    h = conv(h, W4, b4)

    ssum = jax.ops.segment_sum(h, batch, num_segments=B)
    cnt = jax.ops.segment_sum(jnp.ones((N,), h.dtype), batch, num_segments=B)
    p = ssum / jnp.maximum(cnt, 1.0)[:, None]
    return _head(p, H1, hb1, H2, hb2, H3, hb3)


# speedup vs baseline: 10.8543x; 1.2773x over previous
import functools

import jax
import jax.numpy as jnp
from jax import lax
from jax.experimental import pallas as pl
from jax.experimental.pallas import tpu as pltpu
from jax.experimental.pallas import tpu_sc as plsc

N = 10000
E = 160000
D = 256
HID = 256
OUT = 128
B = 8

NC = 2
NS = 16
DH = 128
N_PAD = 10240
RPT = N_PAD // NS

CH = 160
NSUB = CH // 16
NGSUB = CH // 80
EPT = E // NS
NCHUNK = 62
TAIL = EPT - NCHUNK * CH

CH_DEG = 40
EPT_DEG = E // (NC * NS)
NCHUNK_DEG = EPT_DEG // CH_DEG

_MESH = plsc.VectorSubcoreMesh(core_axis_name="c", subcore_axis_name="s")


def _agg_body(h_hbm, src_hbm, dst_hbm, out_hbm,
              isrcA, isrcB, idstA, idstB, rowsA, rowsB, acc,
              semIA, semIB, semGA, semGB, semSA, semSB):
    c = lax.axis_index("c")
    s = lax.axis_index("s")
    base = s * EPT

    pltpu.sync_copy(h_hbm.at[c, pl.ds(s * RPT, RPT)], acc.at[pl.ds(s * RPT, RPT)])
    plsc.subcore_barrier()

    def issue_idx(g, isrc, idst, semI):
        e0 = base + g * CH
        pltpu.async_copy(src_hbm.at[pl.ds(e0, CH)], isrc, semI)
        for j in range(NSUB):
            pltpu.async_copy(dst_hbm.at[pl.ds(e0 + 16 * j, 16)], idst.at[j], semI)

    def wait_idx(g, isrc, idst, semI):
        e0 = base + g * CH
        pltpu.make_async_copy(src_hbm.at[pl.ds(e0, CH)], isrc, semI).wait()
        for j in range(NSUB):
            pltpu.make_async_copy(dst_hbm.at[pl.ds(e0 + 16 * j, 16)], idst.at[j], semI).wait()

    def issue_gather(isrc, rows, semG):
        for j in range(NGSUB):
            pltpu.async_copy(h_hbm.at[c].at[isrc.at[pl.ds(80 * j, 80)]],
                             rows.at[pl.ds(80 * j, 80)], semG)

    def wait_gather(isrc, rows, semG):
        for j in range(NGSUB):
            pltpu.make_async_copy(h_hbm.at[c].at[isrc.at[pl.ds(80 * j, 80)]],
                                  rows.at[pl.ds(80 * j, 80)], semG).wait()

    def issue_scat(idst, rows, semS):
        for j in range(NSUB):
            pltpu.async_copy(rows.at[pl.ds(16 * j, 16)], acc.at[idst.at[j]], semS, add=True)

    def wait_scat(idst, rows, semS):
        for j in range(NSUB):
            pltpu.make_async_copy(rows.at[pl.ds(16 * j, 16)], acc.at[idst.at[j]], semS).wait()

    A = (isrcA, idstA, rowsA, semIA, semGA, semSA)
    Bv = (isrcB, idstB, rowsB, semIB, semGB, semSB)

    issue_idx(0, isrcA, idstA, semIA)
    wait_idx(0, isrcA, idstA, semIA)
    issue_gather(isrcA, rowsA, semGA)

    wait_gather(isrcA, rowsA, semGA)
    issue_scat(idstA, rowsA, semSA)
    issue_idx(1, isrcB, idstB, semIB)
    wait_idx(1, isrcB, idstB, semIB)
    issue_gather(isrcB, rowsB, semGB)
    wait_gather(isrcB, rowsB, semGB)
    issue_scat(idstB, rowsB, semSB)
    wait_scat(idstA, rowsA, semSA)
    issue_idx(2, isrcA, idstA, semIA)
    wait_idx(2, isrcA, idstA, semIA)
    issue_gather(isrcA, rowsA, semGA)

    def body(i, carry):
        g = 2 * i
        wait_gather(isrcA, rowsA, semGA)
        issue_scat(idstA, rowsA, semSA)
        wait_scat(idstB, rowsB, semSB)
        issue_idx(g + 1, isrcB, idstB, semIB)
        wait_idx(g + 1, isrcB, idstB, semIB)
        issue_gather(isrcB, rowsB, semGB)
        wait_gather(isrcB, rowsB, semGB)
        issue_scat(idstB, rowsB, semSB)
        wait_scat(idstA, rowsA, semSA)
        issue_idx(g + 2, isrcA, idstA, semIA)
        wait_idx(g + 2, isrcA, idstA, semIA)
        issue_gather(isrcA, rowsA, semGA)
        return carry

    lax.fori_loop(1, NCHUNK // 2 - 1, body, 0)

    wait_gather(isrcA, rowsA, semGA)
    issue_scat(idstA, rowsA, semSA)
    wait_scat(idstB, rowsB, semSB)
    issue_idx(NCHUNK - 1, isrcB, idstB, semIB)
    wait_idx(NCHUNK - 1, isrcB, idstB, semIB)
    issue_gather(isrcB, rowsB, semGB)
    wait_gather(isrcB, rowsB, semGB)
    issue_scat(idstB, rowsB, semSB)
    wait_scat(idstA, rowsA, semSA)

    e0 = base + NCHUNK * CH
    pltpu.async_copy(src_hbm.at[pl.ds(e0, TAIL)], isrcA.at[pl.ds(0, TAIL)], semIA)
    for j in range(TAIL // 16):
        pltpu.async_copy(dst_hbm.at[pl.ds(e0 + 16 * j, 16)], idstA.at[j], semIA)
    pltpu.make_async_copy(src_hbm.at[pl.ds(e0, TAIL)], isrcA.at[pl.ds(0, TAIL)], semIA).wait()
    for j in range(TAIL // 16):
        pltpu.make_async_copy(dst_hbm.at[pl.ds(e0 + 16 * j, 16)], idstA.at[j], semIA).wait()
    pltpu.async_copy(h_hbm.at[c].at[isrcA.at[pl.ds(0, TAIL)]],
                     rowsA.at[pl.ds(0, TAIL)], semGA)
    pltpu.make_async_copy(h_hbm.at[c].at[isrcA.at[pl.ds(0, TAIL)]],
                          rowsA.at[pl.ds(0, TAIL)], semGA).wait()
    for j in range(TAIL // 16):
        pltpu.async_copy(rowsA.at[pl.ds(16 * j, 16)], acc.at[idstA.at[j]], semSA, add=True)
    for j in range(TAIL // 16):
        pltpu.make_async_copy(rowsA.at[pl.ds(16 * j, 16)], acc.at[idstA.at[j]], semSA).wait()
    wait_scat(idstB, rowsB, semSB)

    plsc.subcore_barrier()
    pltpu.sync_copy(acc.at[pl.ds(s * RPT, RPT)], out_hbm.at[c, pl.ds(s * RPT, RPT)])


def _sc_aggregate(h_split, src, dst):
    k = functools.partial(
        pl.kernel,
        out_type=jax.ShapeDtypeStruct((NC, N_PAD, DH), jnp.float32),
        mesh=_MESH,
        scratch_types=[
            pltpu.VMEM((CH,), jnp.int32),
            pltpu.VMEM((CH,), jnp.int32),
            pltpu.VMEM((NSUB, 16), jnp.int32),
            pltpu.VMEM((NSUB, 16), jnp.int32),
            pltpu.VMEM((CH, DH), jnp.float32),
            pltpu.VMEM((CH, DH), jnp.float32),
            pltpu.VMEM_SHARED((N_PAD, DH), jnp.float32),
            pltpu.SemaphoreType.DMA,
            pltpu.SemaphoreType.DMA,
            pltpu.SemaphoreType.DMA,
            pltpu.SemaphoreType.DMA,
            pltpu.SemaphoreType.DMA,
            pltpu.SemaphoreType.DMA,
        ],
    )(_agg_body)
    return k(h_split, src, dst)


def _deg_body(dst_hbm, zeros_hbm, ones_hbm, out_hbm, ibuf, ones_v, dacc, sem):
    c = lax.axis_index("c")
    s = lax.axis_index("s")

    pltpu.sync_copy(zeros_hbm.at[pl.ds(s * RPT, RPT)], dacc.at[pl.ds(s * RPT, RPT)])
    pltpu.sync_copy(ones_hbm, ones_v)
    plsc.subcore_barrier()

    base = c * (E // NC) + s * EPT_DEG

    def body(g, carry):
        pltpu.async_copy(dst_hbm.at[pl.ds(base + g * CH_DEG, CH_DEG)], ibuf, sem).wait()
        pltpu.sync_copy(ones_v.at[pl.ds(0, CH_DEG)], dacc.at[ibuf], add=True)
        return carry

    lax.fori_loop(0, NCHUNK_DEG, body, 0)

    plsc.subcore_barrier()
    pltpu.sync_copy(dacc.at[pl.ds(s * RPT, RPT)], out_hbm.at[c, pl.ds(s * RPT, RPT)])


def _sc_degree(dst):
    k = functools.partial(
        pl.kernel,
        out_type=jax.ShapeDtypeStruct((NC, N_PAD), jnp.float32),
        mesh=_MESH,
        scratch_types=[
            pltpu.VMEM((CH_DEG,), jnp.int32),
            pltpu.VMEM((128,), jnp.float32),
            pltpu.VMEM_SHARED((N_PAD,), jnp.float32),
            pltpu.SemaphoreType.DMA,
        ],
    )(_deg_body)
    return k(dst, jnp.zeros((N_PAD,), jnp.float32), jnp.ones((128,), jnp.float32))


def _gelu(x):
    c = 0.7978845608028654
    return 0.5 * x * (1.0 + jnp.tanh(c * (x + 0.044715 * x * x * x)))


def _head_body(p_ref, h1_ref, hb1_ref, h2_ref, hb2_ref, h3_ref, hb3_ref, o_ref):
    p = p_ref[...]
    p = _gelu(jnp.dot(p, h1_ref[...], preferred_element_type=jnp.float32) + hb1_ref[...])
    p = _gelu(jnp.dot(p, h2_ref[...], preferred_element_type=jnp.float32) + hb2_ref[...])
    o_ref[...] = jnp.dot(p, h3_ref[...], preferred_element_type=jnp.float32) + hb3_ref[...]


def _head(p, H1, hb1, H2, hb2, H3, hb3):
    return pl.pallas_call(
        _head_body,
        out_shape=jax.ShapeDtypeStruct((B, OUT), jnp.float32),
    )(p, H1, hb1[None, :], H2, hb2[None, :], H3, hb3[None, :])


def kernel(x, edge_index, batch, W1, b1, g1, be1, W2, b2, W3, b3, g3, be3, W4, b4, H1, hb1, H2, hb2, H3, hb3):
    deg_halves = _sc_degree(edge_index[1])
    deg = deg_halves[0, :N] + deg_halves[1, :N] + 1.0
    dinv = deg ** -0.5

    src, dst = edge_index[0], edge_index[1]

    def conv(h, W, b):
        hp = (h @ W) * dinv[:, None]
        hp_split = jnp.stack([hp[:, :DH], hp[:, DH:]])
        hp_split = jnp.pad(hp_split, ((0, 0), (0, N_PAD - N), (0, 0)))
        agg = _sc_aggregate(hp_split, src, dst)
        agg = jnp.concatenate([agg[0, :N], agg[1, :N]], axis=1)
        return agg * dinv[:, None] + b

    def bn(h, g, b):
        m = jnp.mean(h, axis=0)
        v = jnp.var(h, axis=0)
        return (h - m) / jnp.sqrt(v + 1e-5) * g + b

    h = conv(x, W1, b1)
    h = jax.nn.gelu(bn(h, g1, be1))
    h = conv(h, W2, b2)
    h = conv(h, W3, b3)
    h = jax.nn.gelu(bn(h, g3, be3))
    h = conv(h, W4, b4)

    ssum = jax.ops.segment_sum(h, batch, num_segments=B)
    cnt = jax.ops.segment_sum(jnp.ones((N,), h.dtype), batch, num_segments=B)
    p = ssum / jnp.maximum(cnt, 1.0)[:, None]
    return _head(p, H1, hb1, H2, hb2, H3, hb3)


# speedup vs baseline: 11.4113x; 1.0513x over previous
import functools

import jax
import jax.numpy as jnp
from jax import lax
from jax.experimental import pallas as pl
from jax.experimental.pallas import tpu as pltpu
from jax.experimental.pallas import tpu_sc as plsc

N = 10000
E = 160000
D = 256
HID = 256
OUT = 128
B = 8

NC = 2
NS = 16
DH = 128
N_PAD = 10240
RPT = N_PAD // NS

CH = 160
NSUB = CH // 16
NGSUB = CH // 80
EPT = E // NS
NCHUNK = 62
TAIL = EPT - NCHUNK * CH

CH_DEG = 40
EPT_DEG = E // (NC * NS)
NCHUNK_DEG = EPT_DEG // CH_DEG

_MESH = plsc.VectorSubcoreMesh(core_axis_name="c", subcore_axis_name="s")


def _agg_body(h_hbm, src_hbm, dst_hbm, out_hbm,
              isrcA, isrcB, idstA, idstB, rowsA, rowsB, acc,
              semIA, semIB, semGA, semGB, semSA, semSB):
    c = lax.axis_index("c")
    s = lax.axis_index("s")
    base = s * EPT

    pltpu.sync_copy(h_hbm.at[c, pl.ds(s * RPT, RPT)], acc.at[pl.ds(s * RPT, RPT)])
    plsc.subcore_barrier()

    def issue_idx(g, isrc, idst, semI):
        e0 = base + g * CH
        pltpu.async_copy(src_hbm.at[pl.ds(e0, CH)], isrc, semI)
        for j in range(NSUB):
            pltpu.async_copy(dst_hbm.at[pl.ds(e0 + 16 * j, 16)], idst.at[j], semI)

    def wait_idx(g, isrc, idst, semI):
        e0 = base + g * CH
        pltpu.make_async_copy(src_hbm.at[pl.ds(e0, CH)], isrc, semI).wait()
        for j in range(NSUB):
            pltpu.make_async_copy(dst_hbm.at[pl.ds(e0 + 16 * j, 16)], idst.at[j], semI).wait()

    def issue_gather(isrc, rows, semG):
        for j in range(NGSUB):
            pltpu.async_copy(h_hbm.at[c].at[isrc.at[pl.ds(80 * j, 80)]],
                             rows.at[pl.ds(80 * j, 80)], semG)

    def wait_gather(isrc, rows, semG):
        for j in range(NGSUB):
            pltpu.make_async_copy(h_hbm.at[c].at[isrc.at[pl.ds(80 * j, 80)]],
                                  rows.at[pl.ds(80 * j, 80)], semG).wait()

    def issue_scat(idst, rows, semS):
        for j in range(NSUB):
            pltpu.async_copy(rows.at[pl.ds(16 * j, 16)], acc.at[idst.at[j]], semS, add=True)

    def wait_scat(idst, rows, semS):
        for j in range(NSUB):
            pltpu.make_async_copy(rows.at[pl.ds(16 * j, 16)], acc.at[idst.at[j]], semS).wait()

    A = (isrcA, idstA, rowsA, semIA, semGA, semSA)
    Bv = (isrcB, idstB, rowsB, semIB, semGB, semSB)

    issue_idx(0, isrcA, idstA, semIA)
    wait_idx(0, isrcA, idstA, semIA)
    issue_gather(isrcA, rowsA, semGA)

    wait_gather(isrcA, rowsA, semGA)
    issue_scat(idstA, rowsA, semSA)
    issue_idx(1, isrcB, idstB, semIB)
    wait_idx(1, isrcB, idstB, semIB)
    issue_gather(isrcB, rowsB, semGB)
    wait_gather(isrcB, rowsB, semGB)
    issue_scat(idstB, rowsB, semSB)
    wait_scat(idstA, rowsA, semSA)
    issue_idx(2, isrcA, idstA, semIA)
    wait_idx(2, isrcA, idstA, semIA)
    issue_gather(isrcA, rowsA, semGA)

    def body(i, carry):
        g = 2 * i
        wait_gather(isrcA, rowsA, semGA)
        issue_scat(idstA, rowsA, semSA)
        wait_scat(idstB, rowsB, semSB)
        issue_idx(g + 1, isrcB, idstB, semIB)
        wait_idx(g + 1, isrcB, idstB, semIB)
        issue_gather(isrcB, rowsB, semGB)
        wait_gather(isrcB, rowsB, semGB)
        issue_scat(idstB, rowsB, semSB)
        wait_scat(idstA, rowsA, semSA)
        issue_idx(g + 2, isrcA, idstA, semIA)
        wait_idx(g + 2, isrcA, idstA, semIA)
        issue_gather(isrcA, rowsA, semGA)
        return carry

    lax.fori_loop(1, NCHUNK // 2 - 1, body, 0)

    wait_gather(isrcA, rowsA, semGA)
    issue_scat(idstA, rowsA, semSA)
    wait_scat(idstB, rowsB, semSB)
    issue_idx(NCHUNK - 1, isrcB, idstB, semIB)
    wait_idx(NCHUNK - 1, isrcB, idstB, semIB)
    issue_gather(isrcB, rowsB, semGB)
    wait_gather(isrcB, rowsB, semGB)
    issue_scat(idstB, rowsB, semSB)
    wait_scat(idstA, rowsA, semSA)

    e0 = base + NCHUNK * CH
    pltpu.async_copy(src_hbm.at[pl.ds(e0, TAIL)], isrcA.at[pl.ds(0, TAIL)], semIA)
    for j in range(TAIL // 16):
        pltpu.async_copy(dst_hbm.at[pl.ds(e0 + 16 * j, 16)], idstA.at[j], semIA)
    pltpu.make_async_copy(src_hbm.at[pl.ds(e0, TAIL)], isrcA.at[pl.ds(0, TAIL)], semIA).wait()
    for j in range(TAIL // 16):
        pltpu.make_async_copy(dst_hbm.at[pl.ds(e0 + 16 * j, 16)], idstA.at[j], semIA).wait()
    pltpu.async_copy(h_hbm.at[c].at[isrcA.at[pl.ds(0, TAIL)]],
                     rowsA.at[pl.ds(0, TAIL)], semGA)
    pltpu.make_async_copy(h_hbm.at[c].at[isrcA.at[pl.ds(0, TAIL)]],
                          rowsA.at[pl.ds(0, TAIL)], semGA).wait()
    for j in range(TAIL // 16):
        pltpu.async_copy(rowsA.at[pl.ds(16 * j, 16)], acc.at[idstA.at[j]], semSA, add=True)
    for j in range(TAIL // 16):
        pltpu.make_async_copy(rowsA.at[pl.ds(16 * j, 16)], acc.at[idstA.at[j]], semSA).wait()
    wait_scat(idstB, rowsB, semSB)

    plsc.subcore_barrier()
    pltpu.sync_copy(acc.at[pl.ds(s * RPT, RPT)], out_hbm.at[c, pl.ds(s * RPT, RPT)])


def _sc_aggregate(h_split, src, dst):
    k = functools.partial(
        pl.kernel,
        out_type=jax.ShapeDtypeStruct((NC, N_PAD, DH), jnp.float32),
        mesh=_MESH,
        scratch_types=[
            pltpu.VMEM((CH,), jnp.int32),
            pltpu.VMEM((CH,), jnp.int32),
            pltpu.VMEM((NSUB, 16), jnp.int32),
            pltpu.VMEM((NSUB, 16), jnp.int32),
            pltpu.VMEM((CH, DH), jnp.float32),
            pltpu.VMEM((CH, DH), jnp.float32),
            pltpu.VMEM_SHARED((N_PAD, DH), jnp.float32),
            pltpu.SemaphoreType.DMA,
            pltpu.SemaphoreType.DMA,
            pltpu.SemaphoreType.DMA,
            pltpu.SemaphoreType.DMA,
            pltpu.SemaphoreType.DMA,
            pltpu.SemaphoreType.DMA,
        ],
    )(_agg_body)
    return k(h_split, src, dst)


def _deg_body(dst_hbm, zeros_hbm, ones_hbm, out_hbm, ibuf, ones_v, dacc, sem):
    c = lax.axis_index("c")
    s = lax.axis_index("s")

    pltpu.sync_copy(zeros_hbm.at[pl.ds(s * RPT, RPT)], dacc.at[pl.ds(s * RPT, RPT)])
    pltpu.sync_copy(ones_hbm, ones_v)
    plsc.subcore_barrier()

    base = c * (E // NC) + s * EPT_DEG

    def body(g, carry):
        pltpu.async_copy(dst_hbm.at[pl.ds(base + g * CH_DEG, CH_DEG)], ibuf, sem).wait()
        pltpu.sync_copy(ones_v.at[pl.ds(0, CH_DEG)], dacc.at[ibuf], add=True)
        return carry

    lax.fori_loop(0, NCHUNK_DEG, body, 0)

    plsc.subcore_barrier()
    pltpu.sync_copy(dacc.at[pl.ds(s * RPT, RPT)], out_hbm.at[c, pl.ds(s * RPT, RPT)])


def _sc_degree(dst):
    k = functools.partial(
        pl.kernel,
        out_type=jax.ShapeDtypeStruct((NC, N_PAD), jnp.float32),
        mesh=_MESH,
        scratch_types=[
            pltpu.VMEM((CH_DEG,), jnp.int32),
            pltpu.VMEM((128,), jnp.float32),
            pltpu.VMEM_SHARED((N_PAD,), jnp.float32),
            pltpu.SemaphoreType.DMA,
        ],
    )(_deg_body)
    return k(dst, jnp.zeros((N_PAD,), jnp.float32), jnp.ones((128,), jnp.float32))


def _gelu(x):
    c = 0.7978845608028654
    return 0.5 * x * (1.0 + jnp.tanh(c * (x + 0.044715 * x * x * x)))


NBLK = 16
BLK = N_PAD // NBLK


def _mm_body(h_ref, w_ref, dinv_ref, o_ref):
    h = h_ref[...]
    prod = jnp.dot(h, w_ref[0], preferred_element_type=jnp.float32) * dinv_ref[...]
    o_ref[0] = prod
    o_ref[1] = jnp.dot(h, w_ref[1], preferred_element_type=jnp.float32) * dinv_ref[...]


def _mm(h, w_split, dinv_col):
    return pl.pallas_call(
        _mm_body,
        grid=(NBLK,),
        in_specs=[
            pl.BlockSpec((BLK, D), lambda i: (i, 0)),
            pl.BlockSpec((2, D, DH), lambda i: (0, 0, 0)),
            pl.BlockSpec((BLK, 1), lambda i: (i, 0)),
        ],
        out_specs=pl.BlockSpec((2, BLK, DH), lambda i: (0, i, 0)),
        out_shape=jax.ShapeDtypeStruct((2, N_PAD, DH), jnp.float32),
    )(h, w_split, dinv_col)


def _mm_bn_body(h_ref, st_ref, g_ref, be_ref, w_ref, dinv_ref, o_ref):
    s1 = st_ref[0]
    s2 = st_ref[1]
    m = s1 / N
    v = s2 / N - m * m
    t = (h_ref[...] - m) / jnp.sqrt(v + 1e-5) * g_ref[...] + be_ref[...]
    t = _gelu(t)
    o_ref[0] = jnp.dot(t, w_ref[0], preferred_element_type=jnp.float32) * dinv_ref[...]
    o_ref[1] = jnp.dot(t, w_ref[1], preferred_element_type=jnp.float32) * dinv_ref[...]


def _mm_bn(a, stats, g, be, w_split, dinv_col):
    return pl.pallas_call(
        _mm_bn_body,
        grid=(NBLK,),
        in_specs=[
            pl.BlockSpec((BLK, D), lambda i: (i, 0)),
            pl.BlockSpec((2, D), lambda i: (0, 0)),
            pl.BlockSpec((1, D), lambda i: (0, 0)),
            pl.BlockSpec((1, D), lambda i: (0, 0)),
            pl.BlockSpec((2, D, DH), lambda i: (0, 0, 0)),
            pl.BlockSpec((BLK, 1), lambda i: (i, 0)),
        ],
        out_specs=pl.BlockSpec((2, BLK, DH), lambda i: (0, i, 0)),
        out_shape=jax.ShapeDtypeStruct((2, N_PAD, DH), jnp.float32),
    )(a, stats, g[None, :], be[None, :], w_split, dinv_col)


def _post_body(agg_ref, dinv_ref, b_ref, o_ref, st_ref, acc_ref):
    i = pl.program_id(0)
    a = jnp.concatenate([agg_ref[0], agg_ref[1]], axis=1) * dinv_ref[...] + b_ref[...]
    o_ref[...] = a

    @pl.when(i == 0)
    def _():
        acc_ref[...] = jnp.zeros_like(acc_ref)

    rowid = i * BLK + lax.broadcasted_iota(jnp.int32, (BLK, 1), 0)
    am = jnp.where(rowid < N, a, 0.0)
    acc_ref[0] += jnp.sum(am, axis=0)
    acc_ref[1] += jnp.sum(am * am, axis=0)

    @pl.when(i == NBLK - 1)
    def _():
        st_ref[...] = acc_ref[...]


def _post(agg, dinv_col, b):
    return pl.pallas_call(
        _post_body,
        grid=(NBLK,),
        in_specs=[
            pl.BlockSpec((2, BLK, DH), lambda i: (0, i, 0)),
            pl.BlockSpec((BLK, 1), lambda i: (i, 0)),
            pl.BlockSpec((1, D), lambda i: (0, 0)),
        ],
        out_specs=[
            pl.BlockSpec((BLK, D), lambda i: (i, 0)),
            pl.BlockSpec((2, D), lambda i: (0, 0)),
        ],
        out_shape=[
            jax.ShapeDtypeStruct((N_PAD, D), jnp.float32),
            jax.ShapeDtypeStruct((2, D), jnp.float32),
        ],
        scratch_shapes=[pltpu.VMEM((2, D), jnp.float32)],
    )(agg, dinv_col, b[None, :])


def _pool_head_body(agg_ref, dinv_ref, b_ref, batch_ref,
                    h1_ref, hb1_ref, h2_ref, hb2_ref, h3_ref, hb3_ref,
                    o_ref, ps_ref, pc_ref):
    i = pl.program_id(0)
    a = jnp.concatenate([agg_ref[0], agg_ref[1]], axis=1) * dinv_ref[...] + b_ref[...]

    @pl.when(i == 0)
    def _():
        ps_ref[...] = jnp.zeros_like(ps_ref)
        pc_ref[...] = jnp.zeros_like(pc_ref)

    seg = batch_ref[...].reshape(1, BLK)
    oh = (seg == lax.broadcasted_iota(jnp.int32, (B, BLK), 0)).astype(jnp.float32)
    ps_ref[...] += jnp.dot(oh, a, preferred_element_type=jnp.float32)
    pc_ref[...] += jnp.sum(oh, axis=1, keepdims=True)

    @pl.when(i == NBLK - 1)
    def _():
        p = ps_ref[...] / jnp.maximum(pc_ref[...], 1.0)
        p = _gelu(jnp.dot(p, h1_ref[...], preferred_element_type=jnp.float32) + hb1_ref[...])
        p = _gelu(jnp.dot(p, h2_ref[...], preferred_element_type=jnp.float32) + hb2_ref[...])
        o_ref[...] = jnp.dot(p, h3_ref[...], preferred_element_type=jnp.float32) + hb3_ref[...]


def _pool_head(agg, dinv_col, b, batch_pad, H1, hb1, H2, hb2, H3, hb3):
    return pl.pallas_call(
        _pool_head_body,
        grid=(NBLK,),
        in_specs=[
            pl.BlockSpec((2, BLK, DH), lambda i: (0, i, 0)),
            pl.BlockSpec((BLK, 1), lambda i: (i, 0)),
            pl.BlockSpec((1, D), lambda i: (0, 0)),
            pl.BlockSpec((BLK, 1), lambda i: (i, 0)),
            pl.BlockSpec((D, D), lambda i: (0, 0)),
            pl.BlockSpec((1, D), lambda i: (0, 0)),
            pl.BlockSpec((D, D), lambda i: (0, 0)),
            pl.BlockSpec((1, D), lambda i: (0, 0)),
            pl.BlockSpec((D, OUT), lambda i: (0, 0)),
            pl.BlockSpec((1, OUT), lambda i: (0, 0)),
        ],
        out_specs=pl.BlockSpec((B, OUT), lambda i: (0, 0)),
        out_shape=jax.ShapeDtypeStruct((B, OUT), jnp.float32),
        scratch_shapes=[pltpu.VMEM((B, D), jnp.float32), pltpu.VMEM((B, 1), jnp.float32)],
    )(agg, dinv_col, b[None, :], batch_pad, H1, hb1[None, :], H2, hb2[None, :], H3, hb3[None, :])


def kernel(x, edge_index, batch, W1, b1, g1, be1, W2, b2, W3, b3, g3, be3, W4, b4, H1, hb1, H2, hb2, H3, hb3):
    deg_halves = _sc_degree(edge_index[1])
    deg = deg_halves[0, :N] + deg_halves[1, :N] + 1.0
    dinv = deg ** -0.5
    dinv_col = jnp.pad(dinv[:, None], ((0, N_PAD - N), (0, 0)))

    src, dst = edge_index[0], edge_index[1]
    x_pad = jnp.pad(x, ((0, N_PAD - N), (0, 0)))
    batch_pad = jnp.pad(batch[:, None], ((0, N_PAD - N), (0, 0)), constant_values=B)

    def wsplit(W):
        return jnp.stack([W[:, :DH], W[:, DH:]])

    hp1 = _mm(x_pad, wsplit(W1), dinv_col)
    agg1 = _sc_aggregate(hp1, src, dst)
    a1, st1 = _post(agg1, dinv_col, b1)
    hp2 = _mm_bn(a1, st1, g1, be1, wsplit(W2), dinv_col)
    agg2 = _sc_aggregate(hp2, src, dst)
    a2, _st2 = _post(agg2, dinv_col, b2)
    hp3 = _mm(a2, wsplit(W3), dinv_col)
    agg3 = _sc_aggregate(hp3, src, dst)
    a3, st3 = _post(agg3, dinv_col, b3)
    hp4 = _mm_bn(a3, st3, g3, be3, wsplit(W4), dinv_col)
    agg4 = _sc_aggregate(hp4, src, dst)
    return _pool_head(agg4, dinv_col, b4, batch_pad, H1, hb1, H2, hb2, H3, hb3)
